# baseline (speedup 1.0000x reference)
import jax
import jax.numpy as jnp
from jax import lax
from jax.experimental import pallas as pl
from jax.experimental.pallas import tpu as pltpu
from jax.experimental.pallas import tpu_sc as plsc

B, N, D = 4, 4096, 1024
ROWS = 256
NW = 16
RPW = N // NW
RCH = 16
LANES = 16


def _tc_body(x_ref, y_ref, o0_ref, o2_ref):
    xb = x_ref[...]
    yb = y_ref[...]

    def sel4(yv, c0, c1, c2, c3):
        return jnp.where(yv == 0, c0,
               jnp.where(yv == 1, c1,
               jnp.where(yv == 2, c2, c3)))

    for b in range(B):
        yv = yb[b]
        o0_ref[b, :, :] = sel4(yv, xb[0], xb[1], xb[2], xb[3])
        o2_ref[b, :, :] = sel4(
            yv,
            jnp.broadcast_to(xb[b, :, 0][:, None], (ROWS, D)),
            jnp.broadcast_to(xb[b, :, 1][:, None], (ROWS, D)),
            jnp.broadcast_to(xb[b, :, 2][:, None], (ROWS, D)),
            jnp.broadcast_to(xb[b, :, 3][:, None], (ROWS, D)),
        )


def _tc_call(x, y):
    grid = (N // ROWS,)
    blk = pl.BlockSpec((B, ROWS, D), lambda i: (0, i, 0))
    out_shape = jax.ShapeDtypeStruct((B, N, D), jnp.float32)
    return pl.pallas_call(
        _tc_body,
        grid=grid,
        in_specs=[blk, blk],
        out_specs=[blk, blk],
        out_shape=[out_shape, out_shape],
    )(x, y)


NCH = RPW // RCH


def _sc_body(xr_hbm, y_hbm, o_hbm, xr_v, y_v, o_v, ysem, osem):
    c = lax.axis_index("c")
    s = lax.axis_index("s")
    wid = s * 2 + c
    row0 = wid * RPW
    col_iota = lax.iota(jnp.int32, LANES)
    sh_nch = NCH.bit_length() - 1
    ntot = B * NCH

    def y_copy(g, p):
        b = lax.shift_right_logical(g, sh_nch)
        k = lax.bitwise_and(g, NCH - 1)
        return pltpu.make_async_copy(
            y_hbm.at[b, pl.ds(row0 + k * RCH, RCH), :], y_v.at[p], ysem.at[p])

    def o_copy(g, p):
        b = lax.shift_right_logical(g, sh_nch)
        k = lax.bitwise_and(g, NCH - 1)
        return pltpu.make_async_copy(
            o_v.at[p], o_hbm.at[b, pl.ds(row0 + k * RCH, RCH), :], osem.at[p])

    def compute(g, p):
        base = lax.shift_left(lax.shift_right_logical(g, sh_nch), 12)

        @plsc.parallel_loop(0, RCH * (D // LANES), 1, unroll=8)
        def _(t):
            i = lax.shift_right_logical(t, 6)
            joff = lax.shift_left(lax.bitwise_and(t, 63), 4)
            yv = y_v[p, i, pl.ds(joff, LANES)]
            idx = base + lax.shift_left(yv, 10) + (col_iota + joff)
            o_v[p, i, pl.ds(joff, LANES)] = plsc.load_gather(xr_v, [idx])

    def pair_body(t, _):
        g0 = t * 2
        y_copy(g0, 0).wait()
        y_copy(g0 + 1, 1).start()

        @pl.when(t > 0)
        def _():
            o_copy(g0 - 2, 0).wait()
        compute(g0, 0)
        o_copy(g0, 0).start()

        y_copy(g0 + 1, 1).wait()

        @pl.when(t + 1 < ntot // 2)
        def _():
            y_copy(g0 + 2, 0).start()

        @pl.when(t > 0)
        def _():
            o_copy(g0 - 1, 1).wait()
        compute(g0 + 1, 1)
        o_copy(g0 + 1, 1).start()
        return 0

    @pl.when(wid < NW)
    def _run():
        pltpu.sync_copy(xr_hbm, xr_v)
        y_copy(0, 0).start()
        lax.fori_loop(0, ntot // 2, pair_body, 0)
        o_copy(ntot - 2, 0).wait()
        o_copy(ntot - 1, 1).wait()


def _sc_call(xr, y):
    mesh = plsc.VectorSubcoreMesh(core_axis_name="c", subcore_axis_name="s")
    return pl.kernel(
        _sc_body,
        out_type=jax.ShapeDtypeStruct((B, N, D), jnp.float32),
        mesh=mesh,
        scratch_types=[
            pltpu.VMEM((B * 4 * D,), jnp.float32),
            pltpu.VMEM((2, RCH, D), jnp.int32),
            pltpu.VMEM((2, RCH, D), jnp.float32),
            pltpu.SemaphoreType.DMA((2,)),
            pltpu.SemaphoreType.DMA((2,)),
        ],
        compiler_params=pltpu.CompilerParams(needs_layout_passes=False),
    )(xr, y)


def kernel(x, y):
    xr = x[:, 0:4, :].reshape(B * 4 * D)
    o1 = _sc_call(xr, y)
    o0, o2 = _tc_call(x, y)
    return (o0, o1, o2)


# speedup vs baseline: 1.1717x; 1.1717x over previous
import jax
import jax.numpy as jnp
from jax import lax
from jax.experimental import pallas as pl
from jax.experimental.pallas import tpu as pltpu
from jax.experimental.pallas import tpu_sc as plsc

B, N, D = 4, 4096, 1024
ROWS = 256
NW = 32
RPW = N // NW
RCH = 16
LANES = 16


def _tc_body(x_ref, y_ref, o0_ref, o2_ref):
    xb = x_ref[...]
    yb = y_ref[...]

    def sel4(yv, c0, c1, c2, c3):
        return jnp.where(yv == 0, c0,
               jnp.where(yv == 1, c1,
               jnp.where(yv == 2, c2, c3)))

    for b in range(B):
        yv = yb[b]
        o0_ref[b, :, :] = sel4(yv, xb[0], xb[1], xb[2], xb[3])
        o2_ref[b, :, :] = sel4(
            yv,
            jnp.broadcast_to(xb[b, :, 0][:, None], (ROWS, D)),
            jnp.broadcast_to(xb[b, :, 1][:, None], (ROWS, D)),
            jnp.broadcast_to(xb[b, :, 2][:, None], (ROWS, D)),
            jnp.broadcast_to(xb[b, :, 3][:, None], (ROWS, D)),
        )


def _tc_call(x, y):
    grid = (N // ROWS,)
    blk = pl.BlockSpec((B, ROWS, D), lambda i: (0, i, 0))
    out_shape = jax.ShapeDtypeStruct((B, N, D), jnp.float32)
    return pl.pallas_call(
        _tc_body,
        grid=grid,
        in_specs=[blk, blk],
        out_specs=[blk, blk],
        out_shape=[out_shape, out_shape],
    )(x, y)


NCH = RPW // RCH


def _sc_body(xr_hbm, y_hbm, o_hbm, xr_v, y_v, o_v, ysem, osem):
    c = lax.axis_index("c")
    s = lax.axis_index("s")
    wid = s * 2 + c
    row0 = wid * RPW
    col_iota = lax.iota(jnp.int32, LANES)
    sh_nch = NCH.bit_length() - 1
    ntot = B * NCH

    def y_copy(g, p):
        b = lax.shift_right_logical(g, sh_nch)
        k = lax.bitwise_and(g, NCH - 1)
        return pltpu.make_async_copy(
            y_hbm.at[b, pl.ds(row0 + k * RCH, RCH), :], y_v.at[p], ysem.at[p])

    def o_copy(g, p):
        b = lax.shift_right_logical(g, sh_nch)
        k = lax.bitwise_and(g, NCH - 1)
        return pltpu.make_async_copy(
            o_v.at[p], o_hbm.at[b, pl.ds(row0 + k * RCH, RCH), :], osem.at[p])

    def compute(g, p):
        base = lax.shift_left(lax.shift_right_logical(g, sh_nch), 12)

        @plsc.parallel_loop(0, RCH * (D // LANES), 1, unroll=8)
        def _(t):
            i = lax.shift_right_logical(t, 6)
            joff = lax.shift_left(lax.bitwise_and(t, 63), 4)
            yv = y_v[p, i, pl.ds(joff, LANES)]
            idx = base + lax.shift_left(yv, 10) + (col_iota + joff)
            o_v[p, i, pl.ds(joff, LANES)] = plsc.load_gather(xr_v, [idx])

    def pair_body(t, _):
        g0 = t * 2
        y_copy(g0, 0).wait()
        y_copy(g0 + 1, 1).start()

        @pl.when(t > 0)
        def _():
            o_copy(g0 - 2, 0).wait()
        compute(g0, 0)
        o_copy(g0, 0).start()

        y_copy(g0 + 1, 1).wait()

        @pl.when(t + 1 < ntot // 2)
        def _():
            y_copy(g0 + 2, 0).start()

        @pl.when(t > 0)
        def _():
            o_copy(g0 - 1, 1).wait()
        compute(g0 + 1, 1)
        o_copy(g0 + 1, 1).start()
        return 0

    @pl.when(wid < NW)
    def _run():
        pltpu.sync_copy(xr_hbm, xr_v)
        y_copy(0, 0).start()
        lax.fori_loop(0, ntot // 2, pair_body, 0)
        o_copy(ntot - 2, 0).wait()
        o_copy(ntot - 1, 1).wait()


def _sc_call(xr, y):
    mesh = plsc.VectorSubcoreMesh(core_axis_name="c", subcore_axis_name="s")
    return pl.kernel(
        _sc_body,
        out_type=jax.ShapeDtypeStruct((B, N, D), jnp.float32),
        mesh=mesh,
        scratch_types=[
            pltpu.VMEM((B * 4 * D,), jnp.float32),
            pltpu.VMEM((2, RCH, D), jnp.int32),
            pltpu.VMEM((2, RCH, D), jnp.float32),
            pltpu.SemaphoreType.DMA((2,)),
            pltpu.SemaphoreType.DMA((2,)),
        ],
        compiler_params=pltpu.CompilerParams(needs_layout_passes=False),
    )(xr, y)


def kernel(x, y):
    xr = x[:, 0:4, :].reshape(B * 4 * D)
    o1 = _sc_call(xr, y)
    o0, o2 = _tc_call(x, y)
    return (o0, o1, o2)
